# baseline (speedup 1.0000x reference)
import functools

import jax
import jax.numpy as jnp
from jax.experimental import pallas as pl
from jax.experimental.pallas import tpu as pltpu

N_TOK = 16384
N_EMB = 1024
DIM = 2


def _vq_body(nblk, x_ref, wt_ref, esq_ref, w_ref,
             q_ref, perp_ref, qll_ref,
             counts_ref, qll_acc_ref):
    i = pl.program_id(0)
    x = x_ref[...]
    wt = wt_ref[...]
    esq = esq_ref[...]
    t = x.shape[0]
    k = wt.shape[1]

    xsq2 = 0.5 * jnp.sum(x * x, axis=1, keepdims=True)
    xe = jnp.dot(x, wt, preferred_element_type=jnp.float32)
    d2 = (xsq2 + esq) - xe

    mind2 = jnp.min(d2, axis=1, keepdims=True)
    iotaf = jax.lax.broadcasted_iota(jnp.int32, (t, k), 1).astype(jnp.float32)
    idxf = jnp.min(jnp.where(d2 == mind2, iotaf, float(k)),
                   axis=1, keepdims=True)
    onehot = (iotaf == idxf).astype(jnp.float32)

    q = jnp.dot(onehot, w_ref[...], preferred_element_type=jnp.float32)
    q_ref[...] = q

    csum = jnp.sum(onehot, axis=0, keepdims=True)
    qp = jnp.sum((q - x) ** 2)

    @pl.when(i == 0)
    def _init():
        counts_ref[...] = csum
        qll_acc_ref[0] = qp

    @pl.when(i > 0)
    def _acc():
        counts_ref[...] += csum
        qll_acc_ref[0] += qp

    @pl.when(i == nblk - 1)
    def _fin():
        p = counts_ref[...] * (1.0 / N_TOK)
        ent = jnp.sum(p * jnp.log(p + 1e-10), keepdims=True)
        perp_ref[...] = jnp.exp(-ent)
        qll_ref[...] = (qll_acc_ref[0] * (1.0 / (N_TOK * DIM)))[None, None]


@functools.partial(jax.jit, static_argnames=("block_t", "interpret"))
def _vq(inputs, weight, block_t=2048, interpret=False):
    nblk = N_TOK // block_t
    wt = weight.T
    esq = 0.5 * jnp.sum(weight * weight, axis=1)[None, :]
    q, perp, qll = pl.pallas_call(
        functools.partial(_vq_body, nblk),
        grid=(nblk,),
        in_specs=[
            pl.BlockSpec((block_t, DIM), lambda i: (i, 0)),
            pl.BlockSpec((DIM, N_EMB), lambda i: (0, 0)),
            pl.BlockSpec((1, N_EMB), lambda i: (0, 0)),
            pl.BlockSpec((N_EMB, DIM), lambda i: (0, 0)),
        ],
        out_specs=[
            pl.BlockSpec((block_t, DIM), lambda i: (i, 0)),
            pl.BlockSpec((1, 1), lambda i: (0, 0)),
            pl.BlockSpec((1, 1), lambda i: (0, 0)),
        ],
        out_shape=[
            jax.ShapeDtypeStruct((N_TOK, DIM), jnp.float32),
            jax.ShapeDtypeStruct((1, 1), jnp.float32),
            jax.ShapeDtypeStruct((1, 1), jnp.float32),
        ],
        scratch_shapes=[
            pltpu.VMEM((1, N_EMB), jnp.float32),
            pltpu.SMEM((1,), jnp.float32),
        ],
        interpret=interpret,
    )(inputs, wt, esq, weight)
    return q, perp[0, 0], qll[0, 0]


def kernel(inputs, weight, ema_w):
    return _vq(inputs, weight, block_t=8192)


# speedup vs baseline: 1.0483x; 1.0483x over previous
import functools

import jax
import jax.numpy as jnp
from jax.experimental import pallas as pl
from jax.experimental.pallas import tpu as pltpu

N_TOK = 16384
N_EMB = 1024
DIM = 2


def _vq_body(nblk, x_ref, wt_ref, esq_ref, w_ref,
             q_ref, perp_ref, qll_ref,
             counts_ref, qll_acc_ref):
    i = pl.program_id(0)
    x = x_ref[...]
    wt = wt_ref[...]
    esq = esq_ref[...]
    t = x.shape[0]
    k = wt.shape[1]

    xe = jnp.dot(x, wt, preferred_element_type=jnp.float32)
    g = esq - xe

    ming = jnp.min(g, axis=1, keepdims=True)
    iotaf = jax.lax.broadcasted_iota(jnp.int32, (t, k), 1).astype(jnp.float32)
    idxf = jnp.min(jnp.where(g == ming, iotaf, float(k)),
                   axis=1, keepdims=True)
    onehot = (iotaf == idxf).astype(jnp.float32)

    q = jnp.dot(onehot, w_ref[...], preferred_element_type=jnp.float32)
    q_ref[...] = q

    csum = jnp.sum(onehot, axis=0, keepdims=True)
    qp = jnp.sum((q - x) ** 2)

    @pl.when(i == 0)
    def _init():
        counts_ref[...] = csum
        qll_acc_ref[0] = qp

    @pl.when(i > 0)
    def _acc():
        counts_ref[...] += csum
        qll_acc_ref[0] += qp

    @pl.when(i == nblk - 1)
    def _fin():
        p = counts_ref[...] * (1.0 / N_TOK)
        ent = jnp.sum(p * jnp.log(p + 1e-10), keepdims=True)
        perp_ref[...] = jnp.exp(-ent)
        qll_ref[...] = (qll_acc_ref[0] * (1.0 / (N_TOK * DIM)))[None, None]


@functools.partial(jax.jit, static_argnames=("block_t", "interpret"))
def _vq(inputs, weight, block_t=2048, interpret=False):
    nblk = N_TOK // block_t
    wt = weight.T
    esq = 0.5 * jnp.sum(weight * weight, axis=1)[None, :]
    q, perp, qll = pl.pallas_call(
        functools.partial(_vq_body, nblk),
        grid=(nblk,),
        in_specs=[
            pl.BlockSpec((block_t, DIM), lambda i: (i, 0)),
            pl.BlockSpec((DIM, N_EMB), lambda i: (0, 0)),
            pl.BlockSpec((1, N_EMB), lambda i: (0, 0)),
            pl.BlockSpec((N_EMB, DIM), lambda i: (0, 0)),
        ],
        out_specs=[
            pl.BlockSpec((block_t, DIM), lambda i: (i, 0)),
            pl.BlockSpec((1, 1), lambda i: (0, 0)),
            pl.BlockSpec((1, 1), lambda i: (0, 0)),
        ],
        out_shape=[
            jax.ShapeDtypeStruct((N_TOK, DIM), jnp.float32),
            jax.ShapeDtypeStruct((1, 1), jnp.float32),
            jax.ShapeDtypeStruct((1, 1), jnp.float32),
        ],
        scratch_shapes=[
            pltpu.VMEM((1, N_EMB), jnp.float32),
            pltpu.SMEM((1,), jnp.float32),
        ],
        interpret=interpret,
    )(inputs, wt, esq, weight)
    return q, perp[0, 0], qll[0, 0]


def kernel(inputs, weight, ema_w):
    return _vq(inputs, weight, block_t=4096)


# speedup vs baseline: 1.0774x; 1.0277x over previous
import functools

import jax
import jax.numpy as jnp
from jax.experimental import pallas as pl
from jax.experimental.pallas import tpu as pltpu

N_TOK = 16384
N_EMB = 1024
DIM = 2


def _vq_body(nblk, x_ref, wt_ref, esq_ref, w_ref,
             q_ref, perp_ref, qll_ref,
             counts_ref, qll_acc_ref):
    i = pl.program_id(0)
    x = x_ref[...]
    wt = wt_ref[...]
    esq = esq_ref[...]
    t = x.shape[0]
    k = wt.shape[1]

    x0 = x[:, 0:1]
    x1 = x[:, 1:2]
    g = esq - x0 * wt[0:1, :] - x1 * wt[1:2, :]

    ming = jnp.min(g, axis=1, keepdims=True)
    iotaf = jax.lax.broadcasted_iota(jnp.int32, (t, k), 1).astype(jnp.float32)
    idxf = jnp.min(jnp.where(g == ming, iotaf, float(k)),
                   axis=1, keepdims=True)
    onehot = (iotaf == idxf).astype(jnp.float32)

    q = jnp.dot(onehot, w_ref[...], preferred_element_type=jnp.float32)
    q_ref[...] = q

    csum = jnp.sum(onehot, axis=0, keepdims=True)
    qp = jnp.sum((q - x) ** 2)

    @pl.when(i == 0)
    def _init():
        counts_ref[...] = csum
        qll_acc_ref[0] = qp

    @pl.when(i > 0)
    def _acc():
        counts_ref[...] += csum
        qll_acc_ref[0] += qp

    @pl.when(i == nblk - 1)
    def _fin():
        p = counts_ref[...] * (1.0 / N_TOK)
        ent = jnp.sum(p * jnp.log(p + 1e-10), keepdims=True)
        perp_ref[...] = jnp.exp(-ent)
        qll_ref[...] = (qll_acc_ref[0] * (1.0 / (N_TOK * DIM)))[None, None]


@functools.partial(jax.jit, static_argnames=("block_t", "interpret"))
def _vq(inputs, weight, block_t=2048, interpret=False):
    nblk = N_TOK // block_t
    wt = weight.T
    esq = 0.5 * jnp.sum(weight * weight, axis=1)[None, :]
    q, perp, qll = pl.pallas_call(
        functools.partial(_vq_body, nblk),
        grid=(nblk,),
        in_specs=[
            pl.BlockSpec((block_t, DIM), lambda i: (i, 0)),
            pl.BlockSpec((DIM, N_EMB), lambda i: (0, 0)),
            pl.BlockSpec((1, N_EMB), lambda i: (0, 0)),
            pl.BlockSpec((N_EMB, DIM), lambda i: (0, 0)),
        ],
        out_specs=[
            pl.BlockSpec((block_t, DIM), lambda i: (i, 0)),
            pl.BlockSpec((1, 1), lambda i: (0, 0)),
            pl.BlockSpec((1, 1), lambda i: (0, 0)),
        ],
        out_shape=[
            jax.ShapeDtypeStruct((N_TOK, DIM), jnp.float32),
            jax.ShapeDtypeStruct((1, 1), jnp.float32),
            jax.ShapeDtypeStruct((1, 1), jnp.float32),
        ],
        scratch_shapes=[
            pltpu.VMEM((1, N_EMB), jnp.float32),
            pltpu.SMEM((1,), jnp.float32),
        ],
        interpret=interpret,
    )(inputs, wt, esq, weight)
    return q, perp[0, 0], qll[0, 0]


def kernel(inputs, weight, ema_w):
    return _vq(inputs, weight, block_t=4096)
